# baseline (speedup 1.0000x reference)
import functools

import jax
import jax.numpy as jnp
from jax import lax
from jax.experimental import pallas as pl
from jax.experimental.pallas import tpu as pltpu
from jax.experimental.pallas import tpu_sc as plsc

B = 16384
N_CF = 13
N_SF = 26
VOCAB = 33
D = 64

_NC = 2
_NS = 16
_NW = _NC * _NS
_CHUNK = B // _NW


def _mlp_body(cf_ref, w1_ref, b1_ref, w2_ref, b2_ref, out_ref):
    x = cf_ref[...]
    x = jnp.where(jnp.isnan(x), 0.0, x)
    h = jnp.dot(x, w1_ref[...], preferred_element_type=jnp.float32)
    h = jnp.maximum(h + b1_ref[...], 0.0)
    h = jnp.dot(h, w2_ref[...], preferred_element_type=jnp.float32)
    h = jnp.maximum(h + b2_ref[...], 0.0)
    out_ref[...] = h


def _mlp(cfm, W1, b1, W2, b2):
    bs = 2048
    return pl.pallas_call(
        _mlp_body,
        grid=(B // bs,),
        in_specs=[
            pl.BlockSpec((bs, N_CF), lambda i: (i, 0)),
            pl.BlockSpec((N_CF, 2 * N_CF), lambda i: (0, 0)),
            pl.BlockSpec((1, 2 * N_CF), lambda i: (0, 0)),
            pl.BlockSpec((2 * N_CF, D), lambda i: (0, 0)),
            pl.BlockSpec((1, D), lambda i: (0, 0)),
        ],
        out_specs=pl.BlockSpec((bs, D), lambda i: (i, 0)),
        out_shape=jax.ShapeDtypeStruct((B, D), jnp.float32),
    )(cfm, W1.reshape(N_CF, 2 * N_CF), b1.reshape(1, 2 * N_CF),
      W2.reshape(2 * N_CF, D), b2.reshape(1, D))


_NP = N_SF // 2
_PV = VOCAB * VOCAB


def _emb_accumulate(h, sfs, table):
    mesh = plsc.VectorSubcoreMesh(core_axis_name="c", subcore_axis_name="s")

    @functools.partial(
        pl.kernel,
        mesh=mesh,
        compiler_params=pltpu.CompilerParams(use_tc_tiling_on_sc=False),
        out_type=jax.ShapeDtypeStruct((B, D), jnp.float32),
        scratch_types=[
            pltpu.VMEM_SHARED((_NP * _PV, D), jnp.float32),
            pltpu.VMEM((_CHUNK, D), jnp.float32),
            pltpu.VMEM((N_SF, _CHUNK), jnp.int32),
            pltpu.VMEM((_NP, _CHUNK), jnp.int32),
            pltpu.SemaphoreType.DMA,
            pltpu.SemaphoreType.DMA,
        ],
    )
    def k(*refs):
        h_hbm = refs[0]
        sf_refs = refs[1:1 + N_SF]
        t_hbm = refs[1 + N_SF]
        out_hbm = refs[2 + N_SF]
        tab_sp, acc_v, raw_v, po_v, sem_g, sem_i = refs[3 + N_SF:]

        sid = lax.axis_index("s")
        wid = sid * _NC + lax.axis_index("c")
        base = wid * _CHUNK
        rows = pl.ds(base, _CHUNK)

        cp_h = pltpu.async_copy(h_hbm.at[rows], acc_v, sem_i)
        @pl.when(sid == 0)
        def _():
            pltpu.sync_copy(t_hbm, tab_sp)
        idx_cps = [
            pltpu.async_copy(sf_refs[i].at[rows], raw_v.at[i], sem_i)
            for i in range(N_SF)
        ]
        cp_h.wait()
        for cp in idx_cps:
            cp.wait()
        for j in range(_NP):
            for o in range(0, _CHUNK, 16):
                sl = pl.ds(o, 16)
                po_v[j, sl] = (raw_v[2 * j, sl] * VOCAB
                               + raw_v[2 * j + 1, sl] + (_PV * j))
        plsc.subcore_barrier()
        gathers = [
            pltpu.async_copy(tab_sp.at[po_v.at[j]], acc_v, sem_g, add=True)
            for j in range(_NP)
        ]
        for g in gathers:
            g.wait()
        pltpu.sync_copy(acc_v, out_hbm.at[rows])

    return k(h, *sfs, table)


def kernel(cf0, cf1, cf2, cf3, cf4, cf5, cf6, cf7, cf8, cf9, cf10, cf11,
           cf12, sf0, sf1, sf2, sf3, sf4, sf5, sf6, sf7, sf8, sf9, sf10,
           sf11, sf12, sf13, sf14, sf15, sf16, sf17, sf18, sf19, sf20,
           sf21, sf22, sf23, sf24, sf25, W1, b1, W2, b2, emb0, emb1, emb2,
           emb3, emb4, emb5, emb6, emb7, emb8, emb9, emb10, emb11, emb12,
           emb13, emb14, emb15, emb16, emb17, emb18, emb19, emb20, emb21,
           emb22, emb23, emb24, emb25):
    cfs = [cf0, cf1, cf2, cf3, cf4, cf5, cf6, cf7, cf8, cf9, cf10, cf11,
           cf12]
    sfs = [sf0, sf1, sf2, sf3, sf4, sf5, sf6, sf7, sf8, sf9, sf10, sf11,
           sf12, sf13, sf14, sf15, sf16, sf17, sf18, sf19, sf20, sf21,
           sf22, sf23, sf24, sf25]
    embs = [emb0, emb1, emb2, emb3, emb4, emb5, emb6, emb7, emb8, emb9,
            emb10, emb11, emb12, emb13, emb14, emb15, emb16, emb17, emb18,
            emb19, emb20, emb21, emb22, emb23, emb24, emb25]
    cfm = jnp.stack(cfs, axis=1)
    table = jnp.concatenate(
        [(embs[2 * j][:, None, :] + embs[2 * j + 1][None, :, :])
         .reshape(_PV, D) for j in range(_NP)], axis=0)
    h = _mlp(cfm, W1, b1, W2, b2)
    return _emb_accumulate(h, sfs, table)


# speedup vs baseline: 1.4783x; 1.4783x over previous
import functools

import jax
import jax.numpy as jnp
from jax import lax
from jax.experimental import pallas as pl
from jax.experimental.pallas import tpu as pltpu
from jax.experimental.pallas import tpu_sc as plsc

B = 16384
N_CF = 13
N_SF = 26
VOCAB = 33
D = 64

_NC = 2
_NS = 16
_NW = _NC * _NS
_CHUNK = B // _NW


def _mlp_body(cf_ref, w1_ref, b1_ref, w2_ref, b2_ref, out_ref):
    x = cf_ref[...]
    x = jnp.where(jnp.isnan(x), 0.0, x)
    h = jnp.dot(x, w1_ref[...], preferred_element_type=jnp.float32)
    h = jnp.maximum(h + b1_ref[...], 0.0)
    h = jnp.dot(h, w2_ref[...], preferred_element_type=jnp.float32)
    h = jnp.maximum(h + b2_ref[...], 0.0)
    out_ref[...] = h.astype(jnp.bfloat16)


def _mlp(cfm, W1, b1, W2, b2):
    bs = 2048
    return pl.pallas_call(
        _mlp_body,
        grid=(B // bs,),
        in_specs=[
            pl.BlockSpec((bs, N_CF), lambda i: (i, 0)),
            pl.BlockSpec((N_CF, 2 * N_CF), lambda i: (0, 0)),
            pl.BlockSpec((1, 2 * N_CF), lambda i: (0, 0)),
            pl.BlockSpec((2 * N_CF, D), lambda i: (0, 0)),
            pl.BlockSpec((1, D), lambda i: (0, 0)),
        ],
        out_specs=pl.BlockSpec((bs, D), lambda i: (i, 0)),
        out_shape=jax.ShapeDtypeStruct((B, D), jnp.bfloat16),
    )(cfm, W1.reshape(N_CF, 2 * N_CF), b1.reshape(1, 2 * N_CF),
      W2.reshape(2 * N_CF, D), b2.reshape(1, D))


def _emb_accumulate(h, sfs, table):
    mesh = plsc.VectorSubcoreMesh(core_axis_name="c", subcore_axis_name="s")

    @functools.partial(
        pl.kernel,
        mesh=mesh,
        compiler_params=pltpu.CompilerParams(use_tc_tiling_on_sc=False),
        out_type=jax.ShapeDtypeStruct((B, D), jnp.bfloat16),
        scratch_types=[
            pltpu.VMEM_SHARED((N_SF * VOCAB, D), jnp.bfloat16),
            pltpu.VMEM((_CHUNK, D), jnp.bfloat16),
            pltpu.VMEM((N_SF, _CHUNK), jnp.int32),
            pltpu.SemaphoreType.DMA,
            pltpu.SemaphoreType.DMA,
        ],
    )
    def k(*refs):
        h_hbm = refs[0]
        sf_refs = refs[1:1 + N_SF]
        t_hbm = refs[1 + N_SF]
        out_hbm = refs[2 + N_SF]
        tab_sp, acc_v, raw_v, sem_g, sem_i = refs[3 + N_SF:]

        sid = lax.axis_index("s")
        wid = sid * _NC + lax.axis_index("c")
        base = wid * _CHUNK
        rows = pl.ds(base, _CHUNK)

        cp_h = pltpu.async_copy(h_hbm.at[rows], acc_v, sem_i)
        @pl.when(sid == 0)
        def _():
            pltpu.sync_copy(t_hbm, tab_sp)
        idx_cps = [
            pltpu.async_copy(sf_refs[i].at[rows], raw_v.at[i], sem_i)
            for i in range(N_SF)
        ]
        cp_h.wait()
        for cp in idx_cps:
            cp.wait()
        plsc.subcore_barrier()
        gathers = [
            pltpu.async_copy(
                tab_sp.at[pl.ds(VOCAB * i, VOCAB)].at[raw_v.at[i]],
                acc_v, sem_g, add=True)
            for i in range(N_SF)
        ]
        for g in gathers:
            g.wait()
        pltpu.sync_copy(acc_v, out_hbm.at[rows])

    return k(h, *sfs, table)


def kernel(cf0, cf1, cf2, cf3, cf4, cf5, cf6, cf7, cf8, cf9, cf10, cf11,
           cf12, sf0, sf1, sf2, sf3, sf4, sf5, sf6, sf7, sf8, sf9, sf10,
           sf11, sf12, sf13, sf14, sf15, sf16, sf17, sf18, sf19, sf20,
           sf21, sf22, sf23, sf24, sf25, W1, b1, W2, b2, emb0, emb1, emb2,
           emb3, emb4, emb5, emb6, emb7, emb8, emb9, emb10, emb11, emb12,
           emb13, emb14, emb15, emb16, emb17, emb18, emb19, emb20, emb21,
           emb22, emb23, emb24, emb25):
    cfs = [cf0, cf1, cf2, cf3, cf4, cf5, cf6, cf7, cf8, cf9, cf10, cf11,
           cf12]
    sfs = [sf0, sf1, sf2, sf3, sf4, sf5, sf6, sf7, sf8, sf9, sf10, sf11,
           sf12, sf13, sf14, sf15, sf16, sf17, sf18, sf19, sf20, sf21,
           sf22, sf23, sf24, sf25]
    embs = [emb0, emb1, emb2, emb3, emb4, emb5, emb6, emb7, emb8, emb9,
            emb10, emb11, emb12, emb13, emb14, emb15, emb16, emb17, emb18,
            emb19, emb20, emb21, emb22, emb23, emb24, emb25]
    cfm = jnp.stack(cfs, axis=1)
    table = jnp.concatenate(embs, axis=0).astype(jnp.bfloat16)
    h = _mlp(cfm, W1, b1, W2, b2)
    return _emb_accumulate(h, sfs, table).astype(jnp.float32)
